# baseline (speedup 1.0000x reference)
import functools

import jax
import jax.numpy as jnp
from jax import lax
from jax.experimental import pallas as pl
from jax.experimental.pallas import tpu as pltpu
from jax.experimental.pallas import tpu_sc as plsc

MIN_MZ = 0.0
MAX_MZ = 2000.0
BIN_WIDTH = 1.0
NUM_BINS = int((MAX_MZ - MIN_MZ) / BIN_WIDTH)

NC = 2
NS = 16
L = 16
NW = NC * NS

CHUNK = 8
UNROLL = 8

_KEY_SHIFT = 14
_MBITS = 23 - _KEY_SHIFT
_K0 = (127 - 10) << _MBITS
_KN = (((127 + 14) + 1) << _MBITS) - 1
_LUT_N = _KN - _K0 + 1


def _build_sqrt_lut():
    import numpy as np

    keys = np.arange(_K0, _KN + 1, dtype=np.int64)
    lo = (keys << _KEY_SHIFT).astype(np.uint32).view(np.float32)
    hi = ((keys + 1) << _KEY_SHIFT).astype(np.uint32).view(np.float32)
    mid = np.sqrt(lo.astype(np.float64) * hi.astype(np.float64))
    return np.sqrt(mid).astype(np.float32)


_SQRT_LUT = _build_sqrt_lut()


NSPLIT = 4


def kernel(mz, intensities):
    Bfull, P = mz.shape
    assert Bfull % NSPLIT == 0
    B = Bfull // NSPLIT
    assert B % (NW * 2 * CHUNK) == 0 and P % (L * UNROLL) == 0
    rows_per_w = B // NW
    chunks_per_w = rows_per_w // CHUNK
    pairs_per_w = chunks_per_w // 2

    ibits = lax.bitcast_convert_type(intensities, jnp.int32)
    lut = jnp.asarray(_SQRT_LUT)

    mesh = plsc.VectorSubcoreMesh(core_axis_name="c", subcore_axis_name="s")

    @functools.partial(
        pl.kernel,
        out_type=jax.ShapeDtypeStruct((B, NUM_BINS), jnp.float32),
        mesh=mesh,
        compiler_params=pltpu.CompilerParams(
            needs_layout_passes=False, use_tc_tiling_on_sc=True),
        scratch_types=[
            pltpu.VMEM((CHUNK, P), jnp.float32),
            pltpu.VMEM((CHUNK, P), jnp.float32),
            pltpu.VMEM((CHUNK, P), jnp.int32),
            pltpu.VMEM((CHUNK, P), jnp.int32),
            pltpu.VMEM((CHUNK, NUM_BINS), jnp.float32),
            pltpu.VMEM((CHUNK, NUM_BINS), jnp.float32),
            pltpu.VMEM((_LUT_N,), jnp.float32),
            pltpu.SemaphoreType.DMA,
            pltpu.SemaphoreType.DMA,
            pltpu.SemaphoreType.DMA,
            pltpu.SemaphoreType.DMA,
        ],
    )
    def run(mz_hbm, bits_hbm, lut_hbm, out_hbm,
            mz_v0, mz_v1, bits_v0, bits_v1, hist0, hist1, lut_v,
            is0, is1, os0, os1):
        wid = lax.axis_index("s") * NC + lax.axis_index("c")
        row0 = wid * rows_per_w
        pltpu.sync_copy(lut_hbm, lut_v)
        zeros = jnp.zeros((L,), jnp.float32)

        def start_in(base, mz_v, bits_v, sem):
            pltpu.async_copy(mz_hbm.at[pl.ds(base, CHUNK)], mz_v, sem)
            pltpu.async_copy(bits_hbm.at[pl.ds(base, CHUNK)], bits_v, sem)

        def wait_in(mz_v, bits_v, sem):
            pltpu.make_async_copy(mz_hbm.at[pl.ds(0, CHUNK)], mz_v, sem).wait()
            pltpu.make_async_copy(
                bits_hbm.at[pl.ds(0, CHUNK)], bits_v, sem).wait()

        def wait_out(hist, sem):
            pltpu.make_async_copy(
                hist, out_hbm.at[pl.ds(0, CHUNK)], sem).wait()

        def compute_chunk(mz_v, bits_v, hist):
            def row_body(r, carry):
                @plsc.parallel_loop(0, NUM_BINS // L, unroll=UNROLL)
                def zero_body(k):
                    hist[r, pl.ds(k * L, L)] = zeros

                rvec = jnp.full((L,), 0, jnp.int32) + r

                @plsc.parallel_loop(0, P // L, unroll=UNROLL)
                def peak_body(j):
                    o = j * L
                    mzv = mz_v[r, pl.ds(o, L)]
                    xb = bits_v[r, pl.ds(o, L)]
                    mask = (mzv >= jnp.float32(MIN_MZ)) & (
                        mzv < jnp.float32(MAX_MZ)
                    )
                    bi = mzv.astype(jnp.int32)
                    bi = jnp.maximum(bi, 0)
                    bi = jnp.minimum(bi, NUM_BINS - 1)
                    key = (xb >> _KEY_SHIFT) - _K0
                    key = jnp.minimum(jnp.maximum(key, 0), _LUT_N - 1)
                    val = plsc.load_gather(lut_v, [key])
                    plsc.addupdate_scatter(hist, [rvec, bi], val, mask=mask)

                return 0

            lax.fori_loop(0, CHUNK, row_body, 0)

        start_in(row0, mz_v0, bits_v0, is0)
        start_in(row0 + CHUNK, mz_v1, bits_v1, is1)

        def pair_body(cc, carry):
            base0 = row0 + cc * (2 * CHUNK)
            base1 = base0 + CHUNK

            wait_in(mz_v0, bits_v0, is0)

            @pl.when(cc > 0)
            def _():
                wait_out(hist0, os0)

            compute_chunk(mz_v0, bits_v0, hist0)
            pltpu.async_copy(hist0, out_hbm.at[pl.ds(base0, CHUNK)], os0)

            @pl.when(cc < pairs_per_w - 1)
            def _():
                start_in(base0 + 2 * CHUNK, mz_v0, bits_v0, is0)

            wait_in(mz_v1, bits_v1, is1)

            @pl.when(cc > 0)
            def _():
                wait_out(hist1, os1)

            compute_chunk(mz_v1, bits_v1, hist1)
            pltpu.async_copy(hist1, out_hbm.at[pl.ds(base1, CHUNK)], os1)

            @pl.when(cc < pairs_per_w - 1)
            def _():
                start_in(base1 + 2 * CHUNK, mz_v1, bits_v1, is1)

            return 0

        lax.fori_loop(0, pairs_per_w, pair_body, 0)
        wait_out(hist0, os0)
        wait_out(hist1, os1)

    outs = [
        run(mz[i * B:(i + 1) * B], ibits[i * B:(i + 1) * B], lut)
        for i in range(NSPLIT)
    ]
    return jnp.concatenate(outs, axis=0)


# speedup vs baseline: 1.7464x; 1.7464x over previous
import functools

import jax
import jax.numpy as jnp
from jax import lax
from jax.experimental import pallas as pl
from jax.experimental.pallas import tpu as pltpu
from jax.experimental.pallas import tpu_sc as plsc

MIN_MZ = 0.0
MAX_MZ = 2000.0
BIN_WIDTH = 1.0
NUM_BINS = int((MAX_MZ - MIN_MZ) / BIN_WIDTH)

NC = 2
NS = 16
L = 16
NW = NC * NS

CHUNK = 8
UNROLL = 8

NTILE = (NUM_BINS + 127) // 128
NPAD = NTILE * 128

_KEY_SHIFT = 14
_MBITS = 23 - _KEY_SHIFT
_K0 = (127 - 10) << _MBITS
_KN = (((127 + 14) + 1) << _MBITS) - 1
_LUT_N = _KN - _K0 + 1


def _build_sqrt_lut():
    import numpy as np

    keys = np.arange(_K0, _KN + 1, dtype=np.int64)
    lo = (keys << _KEY_SHIFT).astype(np.uint32).view(np.float32)
    hi = ((keys + 1) << _KEY_SHIFT).astype(np.uint32).view(np.float32)
    mid = np.sqrt(lo.astype(np.float64) * hi.astype(np.float64))
    return np.sqrt(mid).astype(np.float32)


_SQRT_LUT = _build_sqrt_lut()


def kernel(mz, intensities):
    B, P = mz.shape
    assert B % (NW * 2 * CHUNK) == 0 and P % (L * UNROLL) == 0
    rows_per_w = B // NW
    stripes_per_w = rows_per_w // CHUNK
    pairs_per_w = stripes_per_w // 2
    nstripe = B // CHUNK

    ibits = lax.bitcast_convert_type(intensities, jnp.int32)
    lut = jnp.asarray(_SQRT_LUT)

    mesh = plsc.VectorSubcoreMesh(core_axis_name="c", subcore_axis_name="s")

    @functools.partial(
        pl.kernel,
        out_type=jax.ShapeDtypeStruct((nstripe, NTILE, CHUNK, 128),
                                      jnp.float32),
        mesh=mesh,
        compiler_params=pltpu.CompilerParams(needs_layout_passes=False),
        scratch_types=[
            pltpu.VMEM((CHUNK, P), jnp.float32),
            pltpu.VMEM((CHUNK, P), jnp.float32),
            pltpu.VMEM((CHUNK, P), jnp.int32),
            pltpu.VMEM((CHUNK, P), jnp.int32),
            pltpu.VMEM((NTILE, CHUNK, 128), jnp.float32),
            pltpu.VMEM((NTILE, CHUNK, 128), jnp.float32),
            pltpu.VMEM((_LUT_N,), jnp.float32),
            pltpu.SemaphoreType.DMA,
            pltpu.SemaphoreType.DMA,
            pltpu.SemaphoreType.DMA,
            pltpu.SemaphoreType.DMA,
        ],
    )
    def run(mz_hbm, bits_hbm, lut_hbm, out_hbm,
            mz_v0, mz_v1, bits_v0, bits_v1, hist0, hist1, lut_v,
            is0, is1, os0, os1):
        wid = lax.axis_index("s") * NC + lax.axis_index("c")
        row0 = wid * rows_per_w
        stripe0 = wid * stripes_per_w
        pltpu.sync_copy(lut_hbm, lut_v)
        zeros = jnp.zeros((L,), jnp.float32)

        def start_in(base, mz_v, bits_v, sem):
            pltpu.async_copy(mz_hbm.at[pl.ds(base, CHUNK)], mz_v, sem)
            pltpu.async_copy(bits_hbm.at[pl.ds(base, CHUNK)], bits_v, sem)

        def wait_in(mz_v, bits_v, sem):
            pltpu.make_async_copy(mz_hbm.at[pl.ds(0, CHUNK)], mz_v, sem).wait()
            pltpu.make_async_copy(
                bits_hbm.at[pl.ds(0, CHUNK)], bits_v, sem).wait()

        def wait_out(hist, sem):
            pltpu.make_async_copy(hist, out_hbm.at[0], sem).wait()

        def compute_chunk(mz_v, bits_v, hist):
            @plsc.parallel_loop(0, NTILE * CHUNK * (128 // L), unroll=UNROLL)
            def zero_body(k):
                hist[k >> 6, (k >> 3) & 7, pl.ds((k & 7) * L, L)] = zeros

            def row_body(r, carry):
                rvec = jnp.full((L,), 0, jnp.int32) + r

                @plsc.parallel_loop(0, P // L, unroll=UNROLL)
                def peak_body(j):
                    o = j * L
                    mzv = mz_v[r, pl.ds(o, L)]
                    xb = bits_v[r, pl.ds(o, L)]
                    mask = (mzv >= jnp.float32(MIN_MZ)) & (
                        mzv < jnp.float32(MAX_MZ)
                    )
                    bi = mzv.astype(jnp.int32)
                    bi = jnp.maximum(bi, 0)
                    bi = jnp.minimum(bi, NUM_BINS - 1)
                    key = (xb >> _KEY_SHIFT) - _K0
                    key = jnp.minimum(jnp.maximum(key, 0), _LUT_N - 1)
                    val = plsc.load_gather(lut_v, [key])
                    plsc.addupdate_scatter(
                        hist, [bi >> 7, rvec, bi & 127], val, mask=mask)

                return 0

            lax.fori_loop(0, CHUNK, row_body, 0)

        start_in(row0, mz_v0, bits_v0, is0)
        start_in(row0 + CHUNK, mz_v1, bits_v1, is1)

        def pair_body(cc, carry):
            base0 = row0 + cc * (2 * CHUNK)
            base1 = base0 + CHUNK
            st0 = stripe0 + cc * 2
            st1 = st0 + 1

            wait_in(mz_v0, bits_v0, is0)

            @pl.when(cc > 0)
            def _():
                wait_out(hist0, os0)

            compute_chunk(mz_v0, bits_v0, hist0)
            pltpu.async_copy(hist0, out_hbm.at[st0], os0)

            @pl.when(cc < pairs_per_w - 1)
            def _():
                start_in(base0 + 2 * CHUNK, mz_v0, bits_v0, is0)

            wait_in(mz_v1, bits_v1, is1)

            @pl.when(cc > 0)
            def _():
                wait_out(hist1, os1)

            compute_chunk(mz_v1, bits_v1, hist1)
            pltpu.async_copy(hist1, out_hbm.at[st1], os1)

            @pl.when(cc < pairs_per_w - 1)
            def _():
                start_in(base1 + 2 * CHUNK, mz_v1, bits_v1, is1)

            return 0

        lax.fori_loop(0, pairs_per_w, pair_body, 0)
        wait_out(hist0, os0)
        wait_out(hist1, os1)

    out4d = run(mz, ibits, lut)
    out = out4d.transpose(0, 2, 1, 3).reshape(B, NPAD)
    return out[:, :NUM_BINS]


# speedup vs baseline: 1.9063x; 1.0916x over previous
import functools

import jax
import jax.numpy as jnp
from jax import lax
from jax.experimental import pallas as pl
from jax.experimental.pallas import tpu as pltpu
from jax.experimental.pallas import tpu_sc as plsc

MIN_MZ = 0.0
MAX_MZ = 2000.0
BIN_WIDTH = 1.0
NUM_BINS = int((MAX_MZ - MIN_MZ) / BIN_WIDTH)

NC = 2
NS = 16
L = 16
NW = NC * NS

CHUNK = 8
UNROLL = 8

NTILE = (NUM_BINS + 127) // 128
NPAD = NTILE * 128

_KEY_SHIFT = 14
_MBITS = 23 - _KEY_SHIFT
_K0 = (127 - 10) << _MBITS
_KN = (((127 + 14) + 1) << _MBITS) - 1
_LUT_N = _KN - _K0 + 1


def _build_sqrt_lut():
    import numpy as np

    keys = np.arange(_K0, _KN + 1, dtype=np.int64)
    lo = (keys << _KEY_SHIFT).astype(np.uint32).view(np.float32)
    hi = ((keys + 1) << _KEY_SHIFT).astype(np.uint32).view(np.float32)
    mid = np.sqrt(lo.astype(np.float64) * hi.astype(np.float64))
    return np.sqrt(mid).astype(np.float32)


_SQRT_LUT = _build_sqrt_lut()


def kernel(mz, intensities):
    B, P = mz.shape
    assert B % (NW * 2 * CHUNK) == 0 and P % (L * UNROLL) == 0
    rows_per_w = B // NW
    stripes_per_w = rows_per_w // CHUNK
    pairs_per_w = stripes_per_w // 2
    nstripe = B // CHUNK

    lut = jnp.asarray(_SQRT_LUT)

    mesh = plsc.VectorSubcoreMesh(core_axis_name="c", subcore_axis_name="s")

    @functools.partial(
        pl.kernel,
        out_type=jax.ShapeDtypeStruct((nstripe, NTILE, CHUNK, 128),
                                      jnp.float32),
        mesh=mesh,
        compiler_params=pltpu.CompilerParams(needs_layout_passes=False),
        scratch_types=[
            pltpu.VMEM((CHUNK, P), jnp.float32),
            pltpu.VMEM((CHUNK, P), jnp.float32),
            pltpu.VMEM((CHUNK, P), jnp.float32),
            pltpu.VMEM((CHUNK, P), jnp.float32),
            pltpu.VMEM((NTILE, CHUNK, 128), jnp.float32),
            pltpu.VMEM((NTILE, CHUNK, 128), jnp.float32),
            pltpu.VMEM((_LUT_N,), jnp.float32),
            pltpu.SemaphoreType.DMA,
            pltpu.SemaphoreType.DMA,
            pltpu.SemaphoreType.DMA,
            pltpu.SemaphoreType.DMA,
        ],
    )
    def run(mz_hbm, int_hbm, lut_hbm, out_hbm,
            mz_v0, mz_v1, in_v0, in_v1, hist0, hist1, lut_v,
            is0, is1, os0, os1):
        wid = lax.axis_index("s") * NC + lax.axis_index("c")
        row0 = wid * rows_per_w
        stripe0 = wid * stripes_per_w
        pltpu.sync_copy(lut_hbm, lut_v)
        zeros = jnp.zeros((L,), jnp.float32)

        def start_in(base, mz_v, in_v, sem):
            pltpu.async_copy(mz_hbm.at[pl.ds(base, CHUNK)], mz_v, sem)
            pltpu.async_copy(int_hbm.at[pl.ds(base, CHUNK)], in_v, sem)

        def wait_in(mz_v, in_v, sem):
            pltpu.make_async_copy(mz_hbm.at[pl.ds(0, CHUNK)], mz_v, sem).wait()
            pltpu.make_async_copy(
                int_hbm.at[pl.ds(0, CHUNK)], in_v, sem).wait()

        def wait_out(hist, sem):
            pltpu.make_async_copy(hist, out_hbm.at[0], sem).wait()

        def compute_chunk(mz_v, in_v, hist):
            bits_v = in_v.bitcast(jnp.int32)
            @plsc.parallel_loop(0, NTILE * CHUNK * (128 // L), unroll=UNROLL)
            def zero_body(k):
                hist[k >> 6, (k >> 3) & 7, pl.ds((k & 7) * L, L)] = zeros

            def row_body(r, carry):
                rvec = jnp.full((L,), 0, jnp.int32) + r

                @plsc.parallel_loop(0, P // L, unroll=UNROLL)
                def peak_body(j):
                    o = j * L
                    mzv = mz_v[r, pl.ds(o, L)]
                    xb = bits_v[r, pl.ds(o, L)]
                    mask = (mzv >= jnp.float32(MIN_MZ)) & (
                        mzv < jnp.float32(MAX_MZ)
                    )
                    bi = mzv.astype(jnp.int32)
                    bi = jnp.maximum(bi, 0)
                    bi = jnp.minimum(bi, NUM_BINS - 1)
                    key = (xb >> _KEY_SHIFT) - _K0
                    key = jnp.minimum(jnp.maximum(key, 0), _LUT_N - 1)
                    val = plsc.load_gather(lut_v, [key])
                    plsc.addupdate_scatter(
                        hist, [bi >> 7, rvec, bi & 127], val, mask=mask)

                return 0

            lax.fori_loop(0, CHUNK, row_body, 0)

        start_in(row0, mz_v0, in_v0, is0)
        start_in(row0 + CHUNK, mz_v1, in_v1, is1)

        def pair_body(cc, carry):
            base0 = row0 + cc * (2 * CHUNK)
            base1 = base0 + CHUNK
            st0 = stripe0 + cc * 2
            st1 = st0 + 1

            wait_in(mz_v0, in_v0, is0)

            @pl.when(cc > 0)
            def _():
                wait_out(hist0, os0)

            compute_chunk(mz_v0, in_v0, hist0)
            pltpu.async_copy(hist0, out_hbm.at[st0], os0)

            @pl.when(cc < pairs_per_w - 1)
            def _():
                start_in(base0 + 2 * CHUNK, mz_v0, in_v0, is0)

            wait_in(mz_v1, in_v1, is1)

            @pl.when(cc > 0)
            def _():
                wait_out(hist1, os1)

            compute_chunk(mz_v1, in_v1, hist1)
            pltpu.async_copy(hist1, out_hbm.at[st1], os1)

            @pl.when(cc < pairs_per_w - 1)
            def _():
                start_in(base1 + 2 * CHUNK, mz_v1, in_v1, is1)

            return 0

        lax.fori_loop(0, pairs_per_w, pair_body, 0)
        wait_out(hist0, os0)
        wait_out(hist1, os1)

    out4d = run(mz, intensities, lut)
    out = out4d.transpose(0, 2, 1, 3).reshape(B, NPAD)
    return out[:, :NUM_BINS]
